# baseline (speedup 1.0000x reference)
import functools

import jax
import jax.numpy as jnp
from jax import lax
from jax.experimental import pallas as pl
from jax.experimental.pallas import tpu as pltpu
from jax.experimental.pallas import tpu_sc as plsc

B, S, D = 4096, 50, 64
S1 = S + 1
DT, TR, LN = 8, 8, 128

_info = plsc.get_sparse_core_info()
NC, NSUB = _info.num_cores, _info.num_subcores
NW = NC * NSUB


@functools.partial(
    pl.kernel,
    out_type=jax.ShapeDtypeStruct((S1, DT, NW, TR, LN), jnp.float32),
    mesh=plsc.VectorSubcoreMesh(core_axis_name="c", subcore_axis_name="s"),
    scratch_types=[
        pltpu.VMEM((2, LN), jnp.int32),
        pltpu.VMEM((2, LN, D), jnp.float32),
        pltpu.VMEM((2, DT, TR, LN), jnp.float32),
        pltpu.VMEM((D, LN), jnp.float32),
        pltpu.SemaphoreType.DMA,
        pltpu.SemaphoreType.DMA,
        pltpu.SemaphoreType.DMA,
        pltpu.SemaphoreType.DMA,
        pltpu.SemaphoreType.DMA,
        pltpu.SemaphoreType.DMA,
    ],
    compiler_params=pltpu.CompilerParams(
        use_tc_tiling_on_sc=False, needs_layout_passes=False
    ),
)
def _gather_concat_t(
    img_hbm, cap_hbm, table_hbm, out_hbm,
    cap_v, rows_v, tile_v, img_v,
    cap_sA, cap_sB, g_sA, g_sB, o_sA, o_sB,
):
    wid = lax.axis_index("s") * NC + lax.axis_index("c")
    b0 = wid * LN
    lane = lax.broadcasted_iota(jnp.int32, (16,), 0)
    rowvecs = [lane + g * 16 for g in range(8)]
    cap_sems = (cap_sA, cap_sB)
    g_sems = (g_sA, g_sB)
    o_sems = (o_sA, o_sB)

    def fire_cap(s, p):
        pltpu.async_copy(cap_hbm.at[s, pl.ds(b0, LN)], cap_v.at[p], cap_sems[p])

    def wait_cap(p):
        pltpu.make_async_copy(
            cap_hbm.at[0, pl.ds(b0, LN)], cap_v.at[p], cap_sems[p]
        ).wait()

    def fire_gather(p):
        pltpu.async_copy(table_hbm.at[cap_v.at[p]], rows_v.at[p], g_sems[p])

    def wait_gather(p):
        pltpu.make_async_copy(
            table_hbm.at[pl.ds(0, LN)], rows_v.at[p], g_sems[p]
        ).wait()

    def transpose(p):
        def half(h, carry):
            colbase = jnp.full((16,), 0, jnp.int32) + h * (D // 2)
            dtbase = h * (D // 2 // TR)
            for dd in range(D // 2):
                col = colbase + dd
                dt = dtbase + dd // TR
                for g in range(8):
                    val = plsc.load_gather(rows_v.at[p], [rowvecs[g], col])
                    tile_v[p, dt, dd % TR, pl.ds(g * 16, 16)] = val
            return carry

        lax.fori_loop(0, 2, half, 0)

    def fire_out(s1, p):
        pltpu.async_copy(
            tile_v.at[p], out_hbm.at[s1, pl.ds(0, DT), wid], o_sems[p]
        )

    def drain_out(p):
        pltpu.make_async_copy(
            tile_v.at[p], out_hbm.at[0, pl.ds(0, DT), wid], o_sems[p]
        ).wait()

    def step(k, p, fire_next_gather, fire_next_cap, drain_first):
        if fire_next_gather:
            wait_cap(1 - p)
            fire_gather(1 - p)
        wait_gather(p)
        if fire_next_cap:
            fire_cap(k + 2, p)
        if drain_first:
            drain_out(p)
        transpose(p)
        fire_out(k + 1, p)

    pltpu.sync_copy(img_hbm.at[pl.ds(0, D), pl.ds(b0, LN)], img_v)
    for dt in range(DT):
        pltpu.sync_copy(img_v.at[pl.ds(dt * TR, TR)], out_hbm.at[0, dt, wid])

    fire_cap(0, 0)
    wait_cap(0)
    fire_gather(0)
    fire_cap(1, 1)

    step(0, 0, True, True, False)
    step(1, 1, True, True, False)

    def jbody(j, carry):
        k = 2 * j
        step(k, 0, True, True, True)
        step(k + 1, 1, True, True, True)
        return carry

    lax.fori_loop(1, 24, jbody, 0)

    step(48, 0, True, False, True)
    step(49, 1, False, False, True)
    drain_out(0)
    drain_out(1)


def kernel(image_features, captions, embedding_table):
    img_t = image_features.T
    cap_t = captions.astype(jnp.int32).T
    out5 = _gather_concat_t(img_t, cap_t, embedding_table)
    return out5.transpose(2, 4, 0, 1, 3).reshape(B, S1, D)


# speedup vs baseline: 1.0400x; 1.0400x over previous
import functools

import jax
import jax.numpy as jnp
from jax import lax
from jax.experimental import pallas as pl
from jax.experimental.pallas import tpu as pltpu
from jax.experimental.pallas import tpu_sc as plsc

B, S, D = 4096, 50, 64
S1 = S + 1
DT, TR, LN = 8, 8, 128

_info = plsc.get_sparse_core_info()
NC, NSUB = _info.num_cores, _info.num_subcores
NW = NC * NSUB


@functools.partial(
    pl.kernel,
    out_type=jax.ShapeDtypeStruct((S1, DT, NW, TR, LN), jnp.float32),
    mesh=plsc.VectorSubcoreMesh(core_axis_name="c", subcore_axis_name="s"),
    scratch_types=[
        pltpu.VMEM((2, LN), jnp.int32),
        pltpu.VMEM((2, LN, 2 * D), jnp.float32),
        pltpu.VMEM((2, DT, TR, LN), jnp.float32),
        pltpu.VMEM((D, LN), jnp.float32),
        pltpu.SemaphoreType.DMA,
        pltpu.SemaphoreType.DMA,
        pltpu.SemaphoreType.DMA,
        pltpu.SemaphoreType.DMA,
        pltpu.SemaphoreType.DMA,
        pltpu.SemaphoreType.DMA,
    ],
    compiler_params=pltpu.CompilerParams(
        use_tc_tiling_on_sc=False, needs_layout_passes=False
    ),
)
def _gather_concat_t(
    img_hbm, cap_hbm, table_hbm, out_hbm,
    cap_v, rows_v, tile_v, img_v,
    cap_sA, cap_sB, g_sA, g_sB, o_sA, o_sB,
):
    wid = lax.axis_index("s") * NC + lax.axis_index("c")
    b0 = wid * LN
    lane = lax.broadcasted_iota(jnp.int32, (16,), 0)
    rowvecs = [lane + g * 16 for g in range(8)]
    cap_sems = (cap_sA, cap_sB)
    g_sems = (g_sA, g_sB)
    o_sems = (o_sA, o_sB)

    def fire_cap(s, p):
        pltpu.async_copy(cap_hbm.at[s, pl.ds(b0, LN)], cap_v.at[p], cap_sems[p])

    def wait_cap(p):
        pltpu.make_async_copy(
            cap_hbm.at[0, pl.ds(b0, LN)], cap_v.at[p], cap_sems[p]
        ).wait()

    def fire_gather(p):
        pltpu.async_copy(table_hbm.at[cap_v.at[p]], rows_v.at[p], g_sems[p])

    def wait_gather(p):
        pltpu.make_async_copy(
            table_hbm.at[pl.ds(0, LN)], rows_v.at[p], g_sems[p]
        ).wait()

    def transpose(p):
        def half(h, carry):
            colbase = jnp.full((16,), 0, jnp.int32) + h * (D // 2)
            dtbase = h * (D // 2 // TR)
            for dd in range(D // 2):
                col = colbase + dd
                dt = dtbase + dd // TR
                for g in range(8):
                    val = plsc.load_gather(rows_v.at[p], [rowvecs[g], col])
                    tile_v[p, dt, dd % TR, pl.ds(g * 16, 16)] = val
            return carry

        lax.fori_loop(0, 2, half, 0)

    def fire_out(s1, p):
        pltpu.async_copy(
            tile_v.at[p], out_hbm.at[s1, pl.ds(0, DT), wid], o_sems[p]
        )

    def drain_out(p):
        pltpu.make_async_copy(
            tile_v.at[p], out_hbm.at[0, pl.ds(0, DT), wid], o_sems[p]
        ).wait()

    def step(k, p, fire_next_gather, fire_next_cap, drain_first):
        if fire_next_gather:
            wait_cap(1 - p)
            fire_gather(1 - p)
        wait_gather(p)
        if fire_next_cap:
            fire_cap(k + 2, p)
        if drain_first:
            drain_out(p)
        transpose(p)
        fire_out(k + 1, p)

    pltpu.sync_copy(img_hbm.at[pl.ds(0, D), pl.ds(b0, LN)], img_v)
    for dt in range(DT):
        pltpu.sync_copy(img_v.at[pl.ds(dt * TR, TR)], out_hbm.at[0, dt, wid])

    fire_cap(0, 0)
    wait_cap(0)
    fire_gather(0)
    fire_cap(1, 1)

    step(0, 0, True, True, False)
    step(1, 1, True, True, False)

    def jbody(j, carry):
        k = 2 * j
        step(k, 0, True, True, True)
        step(k + 1, 1, True, True, True)
        return carry

    lax.fori_loop(1, 24, jbody, 0)

    step(48, 0, True, False, True)
    step(49, 1, False, False, True)
    drain_out(0)
    drain_out(1)


def kernel(image_features, captions, embedding_table):
    img_t = image_features.T
    cap_t = captions.astype(jnp.int32).T
    table2 = jnp.pad(embedding_table, ((0, 0), (0, D)))
    out5 = _gather_concat_t(img_t, cap_t, table2)
    return out5.transpose(2, 4, 0, 1, 3).reshape(B, S1, D)


# speedup vs baseline: 1.0766x; 1.0351x over previous
import functools

import jax
import jax.numpy as jnp
from jax import lax
from jax.experimental import pallas as pl
from jax.experimental.pallas import tpu as pltpu
from jax.experimental.pallas import tpu_sc as plsc

B, S, D = 4096, 50, 64
S1 = S + 1
DT, TR, LN = 8, 8, 128
NP = 2
NSTEP = S // NP

_info = plsc.get_sparse_core_info()
NC, NSUB = _info.num_cores, _info.num_subcores
NW = NC * NSUB


@functools.partial(
    pl.kernel,
    out_type=jax.ShapeDtypeStruct((S1, DT, NW, TR, LN), jnp.float32),
    mesh=plsc.VectorSubcoreMesh(core_axis_name="c", subcore_axis_name="s"),
    scratch_types=[
        pltpu.VMEM((2, NP, LN), jnp.int32),
        pltpu.VMEM((2, NP, LN, 2 * D), jnp.float32),
        pltpu.VMEM((2, NP, DT, TR, LN), jnp.float32),
        pltpu.VMEM((D, LN), jnp.float32),
        pltpu.SemaphoreType.DMA,
        pltpu.SemaphoreType.DMA,
        pltpu.SemaphoreType.DMA,
        pltpu.SemaphoreType.DMA,
        pltpu.SemaphoreType.DMA,
        pltpu.SemaphoreType.DMA,
    ],
    compiler_params=pltpu.CompilerParams(
        use_tc_tiling_on_sc=False,
        needs_layout_passes=False,
        disable_bounds_checks=True,
    ),
)
def _gather_concat_t(
    img_hbm, cap_hbm, table_hbm, out_hbm,
    cap_v, rows_v, tile_v, img_v,
    cap_sA, cap_sB, g_sA, g_sB, o_sA, o_sB,
):
    wid = lax.axis_index("s") * NC + lax.axis_index("c")
    b0 = wid * LN
    lane = lax.broadcasted_iota(jnp.int32, (16,), 0)
    rowvecs = [lane + g * 16 for g in range(8)]
    cap_sems = (cap_sA, cap_sB)
    g_sems = (g_sA, g_sB)
    o_sems = (o_sA, o_sB)

    def fire_cap(k, p):
        pltpu.async_copy(
            cap_hbm.at[pl.ds(k * NP, NP), pl.ds(b0, LN)], cap_v.at[p], cap_sems[p]
        )

    def wait_cap(p):
        pltpu.make_async_copy(
            cap_hbm.at[pl.ds(0, NP), pl.ds(b0, LN)], cap_v.at[p], cap_sems[p]
        ).wait()

    def fire_gather(p):
        for si in range(NP):
            pltpu.async_copy(
                table_hbm.at[cap_v.at[p, si]], rows_v.at[p, si], g_sems[p]
            )

    def wait_gather(p):
        for si in range(NP):
            pltpu.make_async_copy(
                table_hbm.at[pl.ds(0, LN)], rows_v.at[p, si], g_sems[p]
            ).wait()

    QD = 8

    def transpose(p):
        def qbody(q, carry):
            si = q // (D // QD)
            h = q - si * (D // QD)
            colbase = jnp.full((16,), 0, jnp.int32) + h * QD
            dtbase = h * (QD // TR)
            for dd in range(QD):
                col = colbase + dd
                dt = dtbase + dd // TR
                for g in range(8):
                    val = plsc.load_gather(rows_v.at[p, si], [rowvecs[g], col])
                    tile_v[p, si, dt, dd % TR, pl.ds(g * 16, 16)] = val
            return carry

        lax.fori_loop(0, NP * (D // QD), qbody, 0)

    def fire_out(k, p):
        for si in range(NP):
            pltpu.async_copy(
                tile_v.at[p, si],
                out_hbm.at[k * NP + 1 + si, pl.ds(0, DT), wid],
                o_sems[p],
            )

    def drain_out(p):
        for si in range(NP):
            pltpu.make_async_copy(
                tile_v.at[p, si], out_hbm.at[0, pl.ds(0, DT), wid], o_sems[p]
            ).wait()

    def step(k, p, fire_next_gather, fire_next_cap, drain_first):
        if fire_next_gather:
            wait_cap(1 - p)
            fire_gather(1 - p)
        wait_gather(p)
        if fire_next_cap:
            fire_cap(k + 2, p)
        if drain_first:
            drain_out(p)
        transpose(p)
        fire_out(k, p)

    pltpu.sync_copy(img_hbm.at[pl.ds(0, D), pl.ds(b0, LN)], img_v)
    for dt in range(DT):
        pltpu.sync_copy(img_v.at[pl.ds(dt * TR, TR)], out_hbm.at[0, dt, wid])

    fire_cap(0, 0)
    wait_cap(0)
    fire_gather(0)
    fire_cap(1, 1)

    step(0, 0, True, True, False)
    step(1, 1, True, True, False)

    def jbody(j, carry):
        k = 2 * j
        step(k, 0, True, True, True)
        step(k + 1, 1, True, True, True)
        return carry

    lax.fori_loop(1, 11, jbody, 0)

    step(22, 0, True, True, True)
    step(23, 1, True, False, True)
    step(24, 0, False, False, True)
    drain_out(0)
    drain_out(1)


def kernel(image_features, captions, embedding_table):
    img_t = image_features.T
    cap_t = captions.astype(jnp.int32).T
    table2 = jnp.pad(embedding_table, ((0, 0), (0, D)))
    out5 = _gather_concat_t(img_t, cap_t, table2)
    return out5.transpose(2, 4, 0, 1, 3).reshape(B, S1, D)
